# baseline (speedup 1.0000x reference)
import functools

import jax
import jax.numpy as jnp
from jax.experimental import pallas as pl
from jax.experimental.pallas import tpu as pltpu

HIDDEN = 64
HEADS = 2
NG = 64
N = 50000
E = 800000


_PB = 1000


def _pool_mlp_body(h_ref, nt_ref, b_ref, w_refs, acc_num, acc_cnt, out_ref):
    i = pl.program_id(0)

    @pl.when(i == 0)
    def _init():
        acc_num[...] = jnp.zeros_like(acc_num)
        acc_cnt[...] = jnp.zeros_like(acc_cnt)

    h = h_ref[...]
    nt = nt_ref[0]
    b = b_ref[0]
    mask = (nt == 0).astype(jnp.float32)
    gids = jax.lax.broadcasted_iota(jnp.int32, (NG, _PB), 0)
    onehot = (b == gids).astype(jnp.float32)
    hm = h * mask.reshape(_PB, 1)
    acc_num[...] += jnp.dot(onehot, hm, preferred_element_type=jnp.float32)
    acc_cnt[...] += jnp.dot(onehot, mask.reshape(_PB, 1),
                            preferred_element_type=jnp.float32)

    @pl.when(i == pl.num_programs(0) - 1)
    def _final():
        (W1, b1, g1, bb1, W2, b2, g2, bb2, W3, b3, g3, bb3, W4, b4) = [
            r[...] for r in w_refs]

        def ln(v, g, bb):
            m = jnp.mean(v, axis=-1, keepdims=True)
            var = jnp.mean((v - m) ** 2, axis=-1, keepdims=True)
            return (v - m) / jnp.sqrt(var + 1e-5) * g + bb

        def lrelu(v):
            return jnp.maximum(v, 0.2 * v)

        pooled = acc_num[...] / jnp.maximum(acc_cnt[...], 1.0)
        z = lrelu(ln(jnp.dot(pooled, W1, preferred_element_type=jnp.float32) + b1[0], g1[0], bb1[0]))
        z = lrelu(ln(jnp.dot(z, W2, preferred_element_type=jnp.float32) + b2[0], g2[0], bb2[0]))
        z = lrelu(ln(jnp.dot(z, W3, preferred_element_type=jnp.float32) + b3[0], g3[0], bb3[0]))
        z = jnp.dot(z, W4, preferred_element_type=jnp.float32) + b4[0]
        out_ref[...] = jnp.broadcast_to(z.reshape(1, NG), (8, NG))


def _pool_mlp(h, node_type, batch, mp):
    nb = N // _PB
    nt3 = node_type.astype(jnp.int32).reshape(nb, 1, _PB)
    b3 = batch.astype(jnp.int32).reshape(nb, 1, _PB)
    ws = [mp['W1'], mp['b1'].reshape(1, -1), mp['g1'].reshape(1, -1), mp['bb1'].reshape(1, -1),
          mp['W2'], mp['b2'].reshape(1, -1), mp['g2'].reshape(1, -1), mp['bb2'].reshape(1, -1),
          mp['W3'], mp['b3'].reshape(1, -1), mp['g3'].reshape(1, -1), mp['bb3'].reshape(1, -1),
          mp['W4'], mp['b4'].reshape(1, -1)]
    w_specs = [pl.BlockSpec(w.shape, functools.partial(lambda nd, i: (0,) * nd, w.ndim))
               for w in ws]

    grid = (nb,)
    out = pl.pallas_call(
        lambda h_ref, nt_ref, b_ref, *rest: _pool_mlp_body(
            h_ref, nt_ref, b_ref, rest[:-3], rest[-2], rest[-1], rest[-3]),
        grid=grid,
        in_specs=[
            pl.BlockSpec((_PB, HIDDEN), lambda i: (i, 0)),
            pl.BlockSpec((1, 1, _PB), lambda i: (i, 0, 0)),
            pl.BlockSpec((1, 1, _PB), lambda i: (i, 0, 0)),
            *w_specs,
        ],
        out_specs=pl.BlockSpec((8, NG), lambda i: (0, 0)),
        out_shape=jax.ShapeDtypeStruct((8, NG), jnp.float32),
        scratch_shapes=[
            pltpu.VMEM((NG, HIDDEN), jnp.float32),
            pltpu.VMEM((NG, 1), jnp.float32),
        ],
    )(h, nt3, b3, *ws)
    return out[0]


def _ln(v, g, b):
    m = jnp.mean(v, axis=-1, keepdims=True)
    var = jnp.mean((v - m) ** 2, axis=-1, keepdims=True)
    return (v - m) / jnp.sqrt(var + 1e-5) * g + b


def _gatv2(h, edge_index, edge_attr, p):
    src = edge_index[0]
    dst = edge_index[1]
    n = h.shape[0]
    xl = (h @ p['Wl'] + p['bl']).reshape(n, HEADS, HIDDEN)
    xr = (h @ p['Wr'] + p['br']).reshape(n, HEADS, HIDDEN)
    e = (edge_attr @ p['We']).reshape(-1, HEADS, HIDDEN)
    xj = xl[src]
    xi = xr[dst]
    m = jax.nn.leaky_relu(xi + xj + e, 0.2)
    alpha = jnp.sum(m * p['att'][None], axis=-1)
    amax = jax.ops.segment_max(alpha, dst, num_segments=n)
    amax = jnp.where(jnp.isfinite(amax), amax, 0.0)
    ex = jnp.exp(alpha - amax[dst])
    den = jax.ops.segment_sum(ex, dst, num_segments=n)
    a = ex / (den[dst] + 1e-16)
    out = jax.ops.segment_sum(xj * a[..., None], dst, num_segments=n)
    return jnp.mean(out, axis=1) + p['bias']


def kernel(x, node_type, edge_index, edge_attr, batch, params):
    dst_s, src_s, ea_s = jax.lax.sort(
        (edge_index[1], edge_index[0], edge_attr[:, 0]), num_keys=1)
    ei_sorted = jnp.stack([src_s, dst_s])
    ea_sorted = ea_s[:, None]
    h = x @ params['vp_W'] + params['vp_b'] + params['type_emb'][node_type]
    for lp, nrm in zip(params['layers'], params['norms']):
        hh = jax.nn.leaky_relu(_gatv2(h, ei_sorted, ea_sorted, lp), 0.2)
        h = _ln(h + hh, nrm['g'], nrm['b'])
    return _pool_mlp(h, node_type, batch, params['mlp'])


# speedup vs baseline: 35.9403x; 35.9403x over previous
import functools

import jax
import jax.numpy as jnp
from jax import lax
from jax.experimental import pallas as pl
from jax.experimental.pallas import tpu as pltpu
from jax.experimental.pallas import tpu_sc as plsc

HIDDEN = 64
HEADS = 2
NG = 64
N = 50000
E = 800000

CN = 192
NCHUNK = (N + CN - 1) // CN
NPAD = NCHUNK * CN
NW = 32
CPW = (NCHUNK + NW - 1) // NW
KE = 120
BUF = 128
F = HEADS * HIDDEN
OFFPAD = 280

_SC_PARAMS = pltpu.CompilerParams(needs_layout_passes=False)


def _edge_body(xl_hbm, xr_hbm, src_hbm, dst_hbm, ea_hbm, offs_hbm, w_hbm,
               out_hbm, xr_tab, acc, outb, xjbuf, srcbuf, dstbuf, eabuf,
               offbuf, wbuf, sem):
    wid = lax.axis_index("s") * 2 + lax.axis_index("c")
    pltpu.sync_copy(offs_hbm, offbuf)
    pltpu.sync_copy(w_hbm, wbuf)
    lanes = lax.iota(jnp.int32, 16)
    z16 = jnp.zeros((16,), jnp.float32)
    neg = jnp.float32(-jnp.inf)
    minit = jnp.where(lanes < HEADS, neg, 0.0)
    wW = [wbuf[pl.ds(g * 16, 16)] for g in range(8)]
    wA = [wbuf[pl.ds(F + g * 16, 16)] for g in range(8)]

    def chunk_step(i, _):
        c = i * NW + wid

        @pl.when(c < NCHUNK)
        def _do():
            c0 = pl.multiple_of(c * CN, CN)

            def init_row(b, _):
                for g in range(8):
                    acc[b, pl.ds(g * 16, 16)] = z16
                acc[b, pl.ds(F, 16)] = minit
                acc[b, pl.ds(F + 16, 16)] = z16
                return _

            lax.fori_loop(0, CN, init_row, 0)
            pltpu.sync_copy(xr_hbm.at[pl.ds(c0, CN)], xr_tab)
            ov = offbuf[pl.ds(c, 16)]
            e0 = ov[0]
            e1 = ov[1]
            nblk = (e1 - e0 + (KE - 1)) // KE

            def blk(j, _):
                eb = e0 + j * KE
                lb = pl.multiple_of((eb // 8) * 8, 8)
                koff = eb - lb
                cnt = jnp.minimum(e1 - eb, KE)
                pltpu.sync_copy(src_hbm.at[pl.ds(lb, BUF)], srcbuf)
                pltpu.sync_copy(dst_hbm.at[pl.ds(lb, BUF)],
                                dstbuf.at[pl.ds(0, BUF)])
                pltpu.sync_copy(ea_hbm.at[pl.ds(lb, BUF)],
                                eabuf.at[pl.ds(0, BUF)])
                pltpu.async_copy(xl_hbm.at[srcbuf], xjbuf, sem).wait()

                def edge(k, _):
                    q = koff + k
                    b = dstbuf[pl.ds(q, 16)][0] - c0
                    ea = eabuf[pl.ds(q, 16)][0]
                    xjs = []
                    p0 = z16
                    p1 = z16
                    for g in range(8):
                        xj = xjbuf[q, pl.ds(g * 16, 16)]
                        xjs.append(xj)
                        t = xr_tab[b, pl.ds(g * 16, 16)] + xj + ea * wW[g]
                        l = jnp.maximum(t, 0.2 * t)
                        if g < 4:
                            p0 = p0 + l * wA[g]
                        else:
                            p1 = p1 + l * wA[g]
                    a0 = jnp.sum(p0, axis=0)
                    a1 = jnp.sum(p1, axis=0)
                    mv = acc[b, pl.ds(F, 16)]
                    sv = acc[b, pl.ds(F + 16, 16)]
                    av = jnp.where(lanes == 0, a0,
                                   jnp.where(lanes == 1, a1, neg))
                    mn = jnp.maximum(mv, av)
                    cv = jnp.exp(mv - mn)
                    wv = jnp.exp(av - mn)
                    acc[b, pl.ds(F, 16)] = mn
                    acc[b, pl.ds(F + 16, 16)] = sv * cv + wv
                    c0s = cv[0]
                    c1s = cv[1]
                    w0 = wv[0]
                    w1 = wv[1]
                    for g in range(4):
                        o = g * 16
                        acc[b, pl.ds(o, 16)] = (
                            acc[b, pl.ds(o, 16)] * c0s + xjs[g] * w0)
                    for g in range(4):
                        o = HIDDEN + g * 16
                        acc[b, pl.ds(o, 16)] = (
                            acc[b, pl.ds(o, 16)] * c1s + xjs[4 + g] * w1)
                    return _

                lax.fori_loop(0, cnt, edge, 0)
                return _

            lax.fori_loop(0, nblk, blk, 0)

            def fin(b, _):
                sv = acc[b, pl.ds(F + 16, 16)]
                rv = 0.5 / (sv + 1e-16)
                r0 = rv[0]
                r1 = rv[1]
                for g in range(4):
                    o = acc[b, pl.ds(g * 16, 16)] * r0 + \
                        acc[b, pl.ds(HIDDEN + g * 16, 16)] * r1
                    outb[b, pl.ds(g * 16, 16)] = o
                return _

            lax.fori_loop(0, CN, fin, 0)
            pltpu.sync_copy(outb, out_hbm.at[pl.ds(c0, CN)])

        return _

    lax.fori_loop(0, CPW, chunk_step, 0)


def _edge_layer_sc(xl_pad, xr_pad, src_pad, dst_pad, ea_pad, offs_pad, wvec):
    mesh = plsc.VectorSubcoreMesh(core_axis_name="c", subcore_axis_name="s")
    fn = pl.kernel(
        _edge_body,
        mesh=mesh,
        compiler_params=_SC_PARAMS,
        out_type=jax.ShapeDtypeStruct((NPAD, HIDDEN), jnp.float32),
        scratch_types=[
            pltpu.VMEM((CN, F), jnp.float32),
            pltpu.VMEM((CN, F + 32), jnp.float32),
            pltpu.VMEM((CN, HIDDEN), jnp.float32),
            pltpu.VMEM((BUF, F), jnp.float32),
            pltpu.VMEM((BUF,), jnp.int32),
            pltpu.VMEM((BUF + 16,), jnp.int32),
            pltpu.VMEM((BUF + 16,), jnp.float32),
            pltpu.VMEM((OFFPAD,), jnp.int32),
            pltpu.VMEM((2 * F,), jnp.float32),
            pltpu.SemaphoreType.DMA,
        ],
    )
    return fn(xl_pad, xr_pad, src_pad, dst_pad, ea_pad, offs_pad, wvec)


_PB = 1000


def _pool_mlp_body(h_ref, nt_ref, b_ref, w_refs, acc_num, acc_cnt, out_ref):
    i = pl.program_id(0)

    @pl.when(i == 0)
    def _init():
        acc_num[...] = jnp.zeros_like(acc_num)
        acc_cnt[...] = jnp.zeros_like(acc_cnt)

    h = h_ref[...]
    nt = nt_ref[0]
    b = b_ref[0]
    mask = (nt == 0).astype(jnp.float32)
    gids = jax.lax.broadcasted_iota(jnp.int32, (NG, _PB), 0)
    onehot = (b == gids).astype(jnp.float32)
    hm = h * mask.reshape(_PB, 1)
    acc_num[...] += jnp.dot(onehot, hm, preferred_element_type=jnp.float32)
    acc_cnt[...] += jnp.dot(onehot, mask.reshape(_PB, 1),
                            preferred_element_type=jnp.float32)

    @pl.when(i == pl.num_programs(0) - 1)
    def _final():
        (W1, b1, g1, bb1, W2, b2, g2, bb2, W3, b3, g3, bb3, W4, b4) = [
            r[...] for r in w_refs]

        def ln(v, g, bb):
            m = jnp.mean(v, axis=-1, keepdims=True)
            var = jnp.mean((v - m) ** 2, axis=-1, keepdims=True)
            return (v - m) / jnp.sqrt(var + 1e-5) * g + bb

        def lrelu(v):
            return jnp.maximum(v, 0.2 * v)

        pooled = acc_num[...] / jnp.maximum(acc_cnt[...], 1.0)
        z = lrelu(ln(jnp.dot(pooled, W1, preferred_element_type=jnp.float32) + b1[0], g1[0], bb1[0]))
        z = lrelu(ln(jnp.dot(z, W2, preferred_element_type=jnp.float32) + b2[0], g2[0], bb2[0]))
        z = lrelu(ln(jnp.dot(z, W3, preferred_element_type=jnp.float32) + b3[0], g3[0], bb3[0]))
        z = jnp.dot(z, W4, preferred_element_type=jnp.float32) + b4[0]
        out_ref[...] = jnp.broadcast_to(z.reshape(1, NG), (8, NG))


def _pool_mlp(h, node_type, batch, mp):
    nb = N // _PB
    nt3 = node_type.astype(jnp.int32).reshape(nb, 1, _PB)
    b3 = batch.astype(jnp.int32).reshape(nb, 1, _PB)
    ws = [mp['W1'], mp['b1'].reshape(1, -1), mp['g1'].reshape(1, -1), mp['bb1'].reshape(1, -1),
          mp['W2'], mp['b2'].reshape(1, -1), mp['g2'].reshape(1, -1), mp['bb2'].reshape(1, -1),
          mp['W3'], mp['b3'].reshape(1, -1), mp['g3'].reshape(1, -1), mp['bb3'].reshape(1, -1),
          mp['W4'], mp['b4'].reshape(1, -1)]
    w_specs = [pl.BlockSpec(w.shape, functools.partial(lambda nd, i: (0,) * nd, w.ndim))
               for w in ws]

    grid = (nb,)
    out = pl.pallas_call(
        lambda h_ref, nt_ref, b_ref, *rest: _pool_mlp_body(
            h_ref, nt_ref, b_ref, rest[:-3], rest[-2], rest[-1], rest[-3]),
        grid=grid,
        in_specs=[
            pl.BlockSpec((_PB, HIDDEN), lambda i: (i, 0)),
            pl.BlockSpec((1, 1, _PB), lambda i: (i, 0, 0)),
            pl.BlockSpec((1, 1, _PB), lambda i: (i, 0, 0)),
            *w_specs,
        ],
        out_specs=pl.BlockSpec((8, NG), lambda i: (0, 0)),
        out_shape=jax.ShapeDtypeStruct((8, NG), jnp.float32),
        scratch_shapes=[
            pltpu.VMEM((NG, HIDDEN), jnp.float32),
            pltpu.VMEM((NG, 1), jnp.float32),
        ],
    )(h, nt3, b3, *ws)
    return out[0]


def _ln(v, g, b):
    m = jnp.mean(v, axis=-1, keepdims=True)
    var = jnp.mean((v - m) ** 2, axis=-1, keepdims=True)
    return (v - m) / jnp.sqrt(var + 1e-5) * g + b


def kernel(x, node_type, edge_index, edge_attr, batch, params):
    dst_s, src_s, ea_s = jax.lax.sort(
        (edge_index[1].astype(jnp.int32), edge_index[0].astype(jnp.int32),
         edge_attr[:, 0]), num_keys=1)
    offs = jnp.searchsorted(dst_s, jnp.arange(NCHUNK + 1, dtype=jnp.int32) * CN
                            ).astype(jnp.int32)
    offs_pad = jnp.full((OFFPAD,), E, jnp.int32).at[:NCHUNK + 1].set(offs)
    zpad_i = jnp.zeros((BUF,), jnp.int32)
    src_pad = jnp.concatenate([src_s, zpad_i])
    dst_pad = jnp.concatenate([dst_s, zpad_i])
    ea_pad = jnp.concatenate([ea_s, jnp.zeros((BUF,), jnp.float32)])

    h = x @ params['vp_W'] + params['vp_b'] + params['type_emb'][node_type]
    rpad = jnp.zeros((NPAD - N, F), jnp.float32)
    for lp, nrm in zip(params['layers'], params['norms']):
        xlw = jnp.concatenate([h @ lp['Wl'] + lp['bl'], rpad])
        xrw = jnp.concatenate([h @ lp['Wr'] + lp['br'], rpad])
        wvec = jnp.concatenate([lp['We'].reshape(F), lp['att'].reshape(F)])
        om = _edge_layer_sc(xlw, xrw, src_pad, dst_pad, ea_pad,
                            offs_pad, wvec)[:N]
        hh = om + lp['bias']
        hh = jnp.maximum(hh, 0.2 * hh)
        h = _ln(h + hh, nrm['g'], nrm['b'])
    return _pool_mlp(h, node_type, batch, params['mlp'])


# speedup vs baseline: 37.0493x; 1.0309x over previous
import functools

import jax
import jax.numpy as jnp
from jax import lax
from jax.experimental import pallas as pl
from jax.experimental.pallas import tpu as pltpu
from jax.experimental.pallas import tpu_sc as plsc

HIDDEN = 64
HEADS = 2
NG = 64
N = 50000
E = 800000

CN = 192
NCHUNK = (N + CN - 1) // CN
NPAD = NCHUNK * CN
NW = 32
CPW = (NCHUNK + NW - 1) // NW
KE = 120
BUF = 128
F = HEADS * HIDDEN
OFFPAD = 280

_SC_PARAMS = pltpu.CompilerParams(needs_layout_passes=False)


def _edge_body(xl_hbm, xr_hbm, src_hbm, dst_hbm, ea_hbm, offs_hbm, w_hbm,
               out_hbm, xr_tab, acc, outb, xjbuf, srcbuf, dstbuf, eabuf,
               offbuf, wbuf, sem):
    wid = lax.axis_index("s") * 2 + lax.axis_index("c")
    pltpu.sync_copy(offs_hbm, offbuf)
    pltpu.sync_copy(w_hbm, wbuf)
    lanes = lax.iota(jnp.int32, 16)
    z16 = jnp.zeros((16,), jnp.float32)
    neg = jnp.float32(-jnp.inf)
    minit = jnp.where(lanes < HEADS, neg, 0.0)
    wW = [wbuf[pl.ds(g * 16, 16)] for g in range(8)]
    wA = [wbuf[pl.ds(F + g * 16, 16)] for g in range(8)]

    def chunk_step(i, _):
        c = i * NW + wid

        @pl.when(c < NCHUNK)
        def _do():
            c0 = pl.multiple_of(c * CN, CN)

            def init_row(b, _):
                for g in range(8):
                    acc[b, pl.ds(g * 16, 16)] = z16
                acc[b, pl.ds(F, 16)] = minit
                acc[b, pl.ds(F + 16, 16)] = z16
                return _

            lax.fori_loop(0, CN, init_row, 0)
            pltpu.sync_copy(xr_hbm.at[pl.ds(c0, CN)], xr_tab)
            ov = offbuf[pl.ds(c, 16)]
            e0 = ov[0]
            e1 = ov[1]
            nblk = (e1 - e0 + (KE - 1)) // KE

            def blk(j, _):
                eb = e0 + j * KE
                lb = pl.multiple_of((eb // 8) * 8, 8)
                koff = eb - lb
                cnt = jnp.minimum(e1 - eb, KE)
                pltpu.sync_copy(src_hbm.at[pl.ds(lb, BUF)], srcbuf)
                pltpu.sync_copy(dst_hbm.at[pl.ds(lb, BUF)],
                                dstbuf.at[pl.ds(0, BUF)])
                pltpu.sync_copy(ea_hbm.at[pl.ds(lb, BUF)],
                                eabuf.at[pl.ds(0, BUF)])
                pltpu.async_copy(xl_hbm.at[srcbuf], xjbuf, sem).wait()

                def edge(k, _):
                    q = koff + k
                    b = dstbuf[pl.ds(q, 16)][0] - c0
                    ea = eabuf[pl.ds(q, 16)][0]
                    xjs = []
                    p0 = z16
                    p1 = z16
                    for g in range(8):
                        xj = xjbuf[q, pl.ds(g * 16, 16)]
                        xjs.append(xj)
                        t = xr_tab[b, pl.ds(g * 16, 16)] + xj + ea * wW[g]
                        l = jnp.maximum(t, 0.2 * t)
                        if g < 4:
                            p0 = p0 + l * wA[g]
                        else:
                            p1 = p1 + l * wA[g]
                    a0 = jnp.sum(p0, axis=0)
                    a1 = jnp.sum(p1, axis=0)
                    mv = acc[b, pl.ds(F, 16)]
                    sv = acc[b, pl.ds(F + 16, 16)]
                    av = jnp.where(lanes == 0, a0,
                                   jnp.where(lanes == 1, a1, neg))
                    mn = jnp.maximum(mv, av)
                    cv = jnp.exp(mv - mn)
                    wv = jnp.exp(av - mn)
                    acc[b, pl.ds(F, 16)] = mn
                    acc[b, pl.ds(F + 16, 16)] = sv * cv + wv
                    c0s = cv[0]
                    c1s = cv[1]
                    w0 = wv[0]
                    w1 = wv[1]
                    for g in range(4):
                        o = g * 16
                        acc[b, pl.ds(o, 16)] = (
                            acc[b, pl.ds(o, 16)] * c0s + xjs[g] * w0)
                    for g in range(4):
                        o = HIDDEN + g * 16
                        acc[b, pl.ds(o, 16)] = (
                            acc[b, pl.ds(o, 16)] * c1s + xjs[4 + g] * w1)
                    return _

                lax.fori_loop(0, cnt, edge, 0)
                return _

            lax.fori_loop(0, nblk, blk, 0)

            def fin(b, _):
                sv = acc[b, pl.ds(F + 16, 16)]
                rv = 0.5 / (sv + 1e-16)
                r0 = rv[0]
                r1 = rv[1]
                for g in range(4):
                    o = acc[b, pl.ds(g * 16, 16)] * r0 + \
                        acc[b, pl.ds(HIDDEN + g * 16, 16)] * r1
                    outb[b, pl.ds(g * 16, 16)] = o
                return _

            lax.fori_loop(0, CN, fin, 0)
            pltpu.sync_copy(outb, out_hbm.at[pl.ds(c0, CN)])

        return _

    lax.fori_loop(0, CPW, chunk_step, 0)


def _edge_layer_sc(xl_pad, xr_pad, src_pad, dst_pad, ea_pad, offs_pad, wvec):
    mesh = plsc.VectorSubcoreMesh(core_axis_name="c", subcore_axis_name="s")
    fn = pl.kernel(
        _edge_body,
        mesh=mesh,
        compiler_params=_SC_PARAMS,
        out_type=jax.ShapeDtypeStruct((NPAD, HIDDEN), jnp.float32),
        scratch_types=[
            pltpu.VMEM((CN, F), jnp.float32),
            pltpu.VMEM((CN, F + 32), jnp.float32),
            pltpu.VMEM((CN, HIDDEN), jnp.float32),
            pltpu.VMEM((BUF, F), jnp.float32),
            pltpu.VMEM((BUF,), jnp.int32),
            pltpu.VMEM((BUF + 16,), jnp.int32),
            pltpu.VMEM((BUF + 16,), jnp.float32),
            pltpu.VMEM((OFFPAD,), jnp.int32),
            pltpu.VMEM((2 * F,), jnp.float32),
            pltpu.SemaphoreType.DMA,
        ],
    )
    return fn(xl_pad, xr_pad, src_pad, dst_pad, ea_pad, offs_pad, wvec)


_EB = 1000
_JB = 576


def _embed_body(x_ref, nt_ref, vpw_ref, vpb_ref, te_ref, out_ref):
    x = x_ref[0].reshape(_EB, 1)
    nt = nt_ref[0].reshape(_EB, 1)
    h = x * vpw_ref[...] + vpb_ref[...]
    t0 = te_ref[0].reshape(1, HIDDEN)
    t1 = te_ref[1].reshape(1, HIDDEN)
    sel = (nt == 0).astype(jnp.float32)
    out_ref[...] = h + sel * t0 + (1.0 - sel) * t1


def _embed(x, node_type, params):
    nb = N // _EB
    x3 = x.reshape(nb, 1, _EB)
    nt3 = node_type.astype(jnp.int32).reshape(nb, 1, _EB)
    return pl.pallas_call(
        _embed_body,
        grid=(nb,),
        in_specs=[
            pl.BlockSpec((1, 1, _EB), lambda i: (i, 0, 0)),
            pl.BlockSpec((1, 1, _EB), lambda i: (i, 0, 0)),
            pl.BlockSpec((1, HIDDEN), lambda i: (0, 0)),
            pl.BlockSpec((1, HIDDEN), lambda i: (0, 0)),
            pl.BlockSpec((2, HIDDEN), lambda i: (0, 0)),
        ],
        out_specs=pl.BlockSpec((_EB, HIDDEN), lambda i: (i, 0)),
        out_shape=jax.ShapeDtypeStruct((N, HIDDEN), jnp.float32),
    )(x3, nt3, params['vp_W'], params['vp_b'].reshape(1, HIDDEN),
      params['type_emb'])


def _proj_body(h_ref, wl_ref, bl_ref, wr_ref, br_ref, xl_ref, xr_ref):
    h = h_ref[...]
    xl_ref[...] = jnp.dot(h, wl_ref[...],
                          preferred_element_type=jnp.float32) + bl_ref[...]
    xr_ref[...] = jnp.dot(h, wr_ref[...],
                          preferred_element_type=jnp.float32) + br_ref[...]


def _proj(h, lp):
    nb = NPAD // _JB
    return pl.pallas_call(
        _proj_body,
        grid=(nb,),
        in_specs=[
            pl.BlockSpec((_JB, HIDDEN), lambda i: (i, 0)),
            pl.BlockSpec((HIDDEN, F), lambda i: (0, 0)),
            pl.BlockSpec((1, F), lambda i: (0, 0)),
            pl.BlockSpec((HIDDEN, F), lambda i: (0, 0)),
            pl.BlockSpec((1, F), lambda i: (0, 0)),
        ],
        out_specs=[pl.BlockSpec((_JB, F), lambda i: (i, 0)),
                   pl.BlockSpec((_JB, F), lambda i: (i, 0))],
        out_shape=[jax.ShapeDtypeStruct((NPAD, F), jnp.float32),
                   jax.ShapeDtypeStruct((NPAD, F), jnp.float32)],
    )(h, lp['Wl'], lp['bl'].reshape(1, F), lp['Wr'], lp['br'].reshape(1, F))


def _post_body(h_ref, om_ref, b_ref, g_ref, bb_ref, out_ref):
    hh = om_ref[...] + b_ref[...]
    hh = jnp.maximum(hh, 0.2 * hh)
    v = h_ref[...] + hh
    m = jnp.mean(v, axis=-1, keepdims=True)
    var = jnp.mean((v - m) ** 2, axis=-1, keepdims=True)
    out_ref[...] = (v - m) / jnp.sqrt(var + 1e-5) * g_ref[...] + bb_ref[...]


def _post(h, om, bias, nrm):
    nb = N // _EB
    return pl.pallas_call(
        _post_body,
        grid=(nb,),
        in_specs=[
            pl.BlockSpec((_EB, HIDDEN), lambda i: (i, 0)),
            pl.BlockSpec((_EB, HIDDEN), lambda i: (i, 0)),
            pl.BlockSpec((1, HIDDEN), lambda i: (0, 0)),
            pl.BlockSpec((1, HIDDEN), lambda i: (0, 0)),
            pl.BlockSpec((1, HIDDEN), lambda i: (0, 0)),
        ],
        out_specs=pl.BlockSpec((_EB, HIDDEN), lambda i: (i, 0)),
        out_shape=jax.ShapeDtypeStruct((N, HIDDEN), jnp.float32),
    )(h, om, bias.reshape(1, HIDDEN), nrm['g'].reshape(1, HIDDEN),
      nrm['b'].reshape(1, HIDDEN))


_PB = 1000


def _pool_mlp_body(h_ref, nt_ref, b_ref, w_refs, acc_num, acc_cnt, out_ref):
    i = pl.program_id(0)

    @pl.when(i == 0)
    def _init():
        acc_num[...] = jnp.zeros_like(acc_num)
        acc_cnt[...] = jnp.zeros_like(acc_cnt)

    h = h_ref[...]
    nt = nt_ref[0]
    b = b_ref[0]
    mask = (nt == 0).astype(jnp.float32)
    gids = jax.lax.broadcasted_iota(jnp.int32, (NG, _PB), 0)
    onehot = (b == gids).astype(jnp.float32)
    hm = h * mask.reshape(_PB, 1)
    acc_num[...] += jnp.dot(onehot, hm, preferred_element_type=jnp.float32)
    acc_cnt[...] += jnp.dot(onehot, mask.reshape(_PB, 1),
                            preferred_element_type=jnp.float32)

    @pl.when(i == pl.num_programs(0) - 1)
    def _final():
        (W1, b1, g1, bb1, W2, b2, g2, bb2, W3, b3, g3, bb3, W4, b4) = [
            r[...] for r in w_refs]

        def ln(v, g, bb):
            m = jnp.mean(v, axis=-1, keepdims=True)
            var = jnp.mean((v - m) ** 2, axis=-1, keepdims=True)
            return (v - m) / jnp.sqrt(var + 1e-5) * g + bb

        def lrelu(v):
            return jnp.maximum(v, 0.2 * v)

        pooled = acc_num[...] / jnp.maximum(acc_cnt[...], 1.0)
        z = lrelu(ln(jnp.dot(pooled, W1, preferred_element_type=jnp.float32) + b1[0], g1[0], bb1[0]))
        z = lrelu(ln(jnp.dot(z, W2, preferred_element_type=jnp.float32) + b2[0], g2[0], bb2[0]))
        z = lrelu(ln(jnp.dot(z, W3, preferred_element_type=jnp.float32) + b3[0], g3[0], bb3[0]))
        z = jnp.dot(z, W4, preferred_element_type=jnp.float32) + b4[0]
        out_ref[...] = jnp.broadcast_to(z.reshape(1, NG), (8, NG))


def _pool_mlp(h, node_type, batch, mp):
    nb = N // _PB
    nt3 = node_type.astype(jnp.int32).reshape(nb, 1, _PB)
    b3 = batch.astype(jnp.int32).reshape(nb, 1, _PB)
    ws = [mp['W1'], mp['b1'].reshape(1, -1), mp['g1'].reshape(1, -1), mp['bb1'].reshape(1, -1),
          mp['W2'], mp['b2'].reshape(1, -1), mp['g2'].reshape(1, -1), mp['bb2'].reshape(1, -1),
          mp['W3'], mp['b3'].reshape(1, -1), mp['g3'].reshape(1, -1), mp['bb3'].reshape(1, -1),
          mp['W4'], mp['b4'].reshape(1, -1)]
    w_specs = [pl.BlockSpec(w.shape, functools.partial(lambda nd, i: (0,) * nd, w.ndim))
               for w in ws]

    grid = (nb,)
    out = pl.pallas_call(
        lambda h_ref, nt_ref, b_ref, *rest: _pool_mlp_body(
            h_ref, nt_ref, b_ref, rest[:-3], rest[-2], rest[-1], rest[-3]),
        grid=grid,
        in_specs=[
            pl.BlockSpec((_PB, HIDDEN), lambda i: (i, 0)),
            pl.BlockSpec((1, 1, _PB), lambda i: (i, 0, 0)),
            pl.BlockSpec((1, 1, _PB), lambda i: (i, 0, 0)),
            *w_specs,
        ],
        out_specs=pl.BlockSpec((8, NG), lambda i: (0, 0)),
        out_shape=jax.ShapeDtypeStruct((8, NG), jnp.float32),
        scratch_shapes=[
            pltpu.VMEM((NG, HIDDEN), jnp.float32),
            pltpu.VMEM((NG, 1), jnp.float32),
        ],
    )(h, nt3, b3, *ws)
    return out[0]


def _ln(v, g, b):
    m = jnp.mean(v, axis=-1, keepdims=True)
    var = jnp.mean((v - m) ** 2, axis=-1, keepdims=True)
    return (v - m) / jnp.sqrt(var + 1e-5) * g + b


def kernel(x, node_type, edge_index, edge_attr, batch, params):
    dst_s, src_s, ea_s = jax.lax.sort(
        (edge_index[1].astype(jnp.int32), edge_index[0].astype(jnp.int32),
         edge_attr[:, 0]), num_keys=1)
    offs = jnp.searchsorted(dst_s, jnp.arange(NCHUNK + 1, dtype=jnp.int32) * CN
                            ).astype(jnp.int32)
    offs_pad = jnp.full((OFFPAD,), E, jnp.int32).at[:NCHUNK + 1].set(offs)
    zpad_i = jnp.zeros((BUF,), jnp.int32)
    src_pad = jnp.concatenate([src_s, zpad_i])
    dst_pad = jnp.concatenate([dst_s, zpad_i])
    ea_pad = jnp.concatenate([ea_s, jnp.zeros((BUF,), jnp.float32)])

    h = _embed(x, node_type, params)
    for lp, nrm in zip(params['layers'], params['norms']):
        xlw, xrw = _proj(h, lp)
        wvec = jnp.concatenate([lp['We'].reshape(F), lp['att'].reshape(F)])
        om = _edge_layer_sc(xlw, xrw, src_pad, dst_pad, ea_pad,
                            offs_pad, wvec)
        h = _post(h, om, lp['bias'], nrm)
    return _pool_mlp(h, node_type, batch, params['mlp'])


# speedup vs baseline: 37.0538x; 1.0001x over previous
import functools

import jax
import jax.numpy as jnp
from jax import lax
from jax.experimental import pallas as pl
from jax.experimental.pallas import tpu as pltpu
from jax.experimental.pallas import tpu_sc as plsc

HIDDEN = 64
HEADS = 2
NG = 64
N = 50000
E = 800000

CN = 192
NCHUNK = (N + CN - 1) // CN
NPAD = NCHUNK * CN
NW = 32
CPW = (NCHUNK + NW - 1) // NW
KE = 120
BUF = 128
F = HEADS * HIDDEN
OFFPAD = 280

_SC_PARAMS = pltpu.CompilerParams(needs_layout_passes=False)


def _edge_body(xl_hbm, xr_hbm, src_hbm, dst_hbm, ea_hbm, offs_hbm, w_hbm,
               out_hbm, xr_tab, acc, outb, xjbuf, srcbuf, dstbuf, eabuf,
               offbuf, wbuf, sem):
    wid = lax.axis_index("s") * 2 + lax.axis_index("c")
    pltpu.sync_copy(offs_hbm, offbuf)
    pltpu.sync_copy(w_hbm, wbuf)
    lanes = lax.iota(jnp.int32, 16)
    z16 = jnp.zeros((16,), jnp.float32)
    neg = jnp.float32(-jnp.inf)
    minit = jnp.where(lanes < HEADS, neg, 0.0)
    wW = [wbuf[pl.ds(g * 16, 16)] for g in range(8)]
    wA = [wbuf[pl.ds(F + g * 16, 16)] for g in range(8)]

    def chunk_step(i, _):
        c = i * NW + wid

        @pl.when(c < NCHUNK)
        def _do():
            c0 = pl.multiple_of(c * CN, CN)

            def init_row(b, _):
                for g in range(8):
                    acc[b, pl.ds(g * 16, 16)] = z16
                acc[b, pl.ds(F, 16)] = minit
                acc[b, pl.ds(F + 16, 16)] = z16
                return _

            lax.fori_loop(0, CN, init_row, 0)
            pltpu.sync_copy(xr_hbm.at[pl.ds(c0, CN)], xr_tab)
            ov = offbuf[pl.ds(c, 16)]
            e0 = ov[0]
            e1 = ov[1]
            nblk = (e1 - e0 + (KE - 1)) // KE

            def blk(j, _):
                eb = e0 + j * KE
                lb = pl.multiple_of((eb // 8) * 8, 8)
                koff = eb - lb
                cnt = jnp.minimum(e1 - eb, KE)
                pltpu.sync_copy(src_hbm.at[pl.ds(lb, BUF)], srcbuf)
                pltpu.sync_copy(dst_hbm.at[pl.ds(lb, BUF)],
                                dstbuf.at[pl.ds(0, BUF)])
                pltpu.sync_copy(ea_hbm.at[pl.ds(lb, BUF)],
                                eabuf.at[pl.ds(0, BUF)])
                pltpu.async_copy(xl_hbm.at[srcbuf], xjbuf, sem).wait()

                def edge(k, _):
                    q = koff + k
                    b = dstbuf[pl.ds(q, 16)][0] - c0
                    ea = eabuf[pl.ds(q, 16)][0]
                    xjs = []
                    p0 = z16
                    p1 = z16
                    for g in range(8):
                        xj = xjbuf[q, pl.ds(g * 16, 16)]
                        xjs.append(xj)
                        t = xr_tab[b, pl.ds(g * 16, 16)] + xj + ea * wW[g]
                        l = jnp.maximum(t, 0.2 * t)
                        if g < 4:
                            p0 = p0 + l * wA[g]
                        else:
                            p1 = p1 + l * wA[g]
                    a0 = jnp.sum(p0, axis=0)
                    a1 = jnp.sum(p1, axis=0)
                    mv = acc[b, pl.ds(F, 16)]
                    sv = acc[b, pl.ds(F + 16, 16)]
                    av = jnp.where(lanes == 0, a0,
                                   jnp.where(lanes == 1, a1, neg))
                    mn = jnp.maximum(mv, av)
                    cv = jnp.exp(mv - mn)
                    wv = jnp.exp(av - mn)
                    acc[b, pl.ds(F, 16)] = mn
                    acc[b, pl.ds(F + 16, 16)] = sv * cv + wv
                    c0s = cv[0]
                    c1s = cv[1]
                    w0 = wv[0]
                    w1 = wv[1]
                    for g in range(4):
                        o = g * 16
                        acc[b, pl.ds(o, 16)] = (
                            acc[b, pl.ds(o, 16)] * c0s + xjs[g] * w0)
                    for g in range(4):
                        o = HIDDEN + g * 16
                        acc[b, pl.ds(o, 16)] = (
                            acc[b, pl.ds(o, 16)] * c1s + xjs[4 + g] * w1)
                    return _

                lax.fori_loop(0, cnt, edge, 0)
                return _

            lax.fori_loop(0, nblk, blk, 0)

            def fin(b, _):
                sv = acc[b, pl.ds(F + 16, 16)]
                rv = 0.5 / (sv + 1e-16)
                r0 = rv[0]
                r1 = rv[1]
                for g in range(4):
                    o = acc[b, pl.ds(g * 16, 16)] * r0 + \
                        acc[b, pl.ds(HIDDEN + g * 16, 16)] * r1
                    outb[b, pl.ds(g * 16, 16)] = o
                return _

            lax.fori_loop(0, CN, fin, 0)
            pltpu.sync_copy(outb, out_hbm.at[pl.ds(c0, CN)])

        return _

    lax.fori_loop(0, CPW, chunk_step, 0)


def _edge_layer_sc(xl_pad, xr_pad, src_pad, dst_pad, ea_pad, offs_pad, wvec):
    mesh = plsc.VectorSubcoreMesh(core_axis_name="c", subcore_axis_name="s")
    fn = pl.kernel(
        _edge_body,
        mesh=mesh,
        compiler_params=_SC_PARAMS,
        out_type=jax.ShapeDtypeStruct((NPAD, HIDDEN), jnp.float32),
        scratch_types=[
            pltpu.VMEM((CN, F), jnp.float32),
            pltpu.VMEM((CN, F + 32), jnp.float32),
            pltpu.VMEM((CN, HIDDEN), jnp.float32),
            pltpu.VMEM((BUF, F), jnp.float32),
            pltpu.VMEM((BUF,), jnp.int32),
            pltpu.VMEM((BUF + 16,), jnp.int32),
            pltpu.VMEM((BUF + 16,), jnp.float32),
            pltpu.VMEM((OFFPAD,), jnp.int32),
            pltpu.VMEM((2 * F,), jnp.float32),
            pltpu.SemaphoreType.DMA,
        ],
    )
    return fn(xl_pad, xr_pad, src_pad, dst_pad, ea_pad, offs_pad, wvec)


_EB = 1000
_JB = 576


def _embed_body(x_ref, nt_ref, vpw_ref, vpb_ref, te_ref, out_ref):
    x = x_ref[0].reshape(_EB, 1)
    nt = nt_ref[0].reshape(_EB, 1)
    h = x * vpw_ref[...] + vpb_ref[...]
    t0 = te_ref[0].reshape(1, HIDDEN)
    t1 = te_ref[1].reshape(1, HIDDEN)
    sel = (nt == 0).astype(jnp.float32)
    out_ref[...] = h + sel * t0 + (1.0 - sel) * t1


def _embed(x, node_type, params):
    nb = N // _EB
    x3 = x.reshape(nb, 1, _EB)
    nt3 = node_type.astype(jnp.int32).reshape(nb, 1, _EB)
    return pl.pallas_call(
        _embed_body,
        grid=(nb,),
        in_specs=[
            pl.BlockSpec((1, 1, _EB), lambda i: (i, 0, 0)),
            pl.BlockSpec((1, 1, _EB), lambda i: (i, 0, 0)),
            pl.BlockSpec((1, HIDDEN), lambda i: (0, 0)),
            pl.BlockSpec((1, HIDDEN), lambda i: (0, 0)),
            pl.BlockSpec((2, HIDDEN), lambda i: (0, 0)),
        ],
        out_specs=pl.BlockSpec((_EB, HIDDEN), lambda i: (i, 0)),
        out_shape=jax.ShapeDtypeStruct((N, HIDDEN), jnp.float32),
    )(x3, nt3, params['vp_W'], params['vp_b'].reshape(1, HIDDEN),
      params['type_emb'])


def _proj_body(h_ref, wl_ref, bl_ref, wr_ref, br_ref, xl_ref, xr_ref):
    h = h_ref[...]
    xl_ref[...] = jnp.dot(h, wl_ref[...],
                          preferred_element_type=jnp.float32) + bl_ref[...]
    xr_ref[...] = jnp.dot(h, wr_ref[...],
                          preferred_element_type=jnp.float32) + br_ref[...]


def _proj(h, lp):
    nb = NPAD // _JB
    return pl.pallas_call(
        _proj_body,
        grid=(nb,),
        in_specs=[
            pl.BlockSpec((_JB, HIDDEN), lambda i: (i, 0)),
            pl.BlockSpec((HIDDEN, F), lambda i: (0, 0)),
            pl.BlockSpec((1, F), lambda i: (0, 0)),
            pl.BlockSpec((HIDDEN, F), lambda i: (0, 0)),
            pl.BlockSpec((1, F), lambda i: (0, 0)),
        ],
        out_specs=[pl.BlockSpec((_JB, F), lambda i: (i, 0)),
                   pl.BlockSpec((_JB, F), lambda i: (i, 0))],
        out_shape=[jax.ShapeDtypeStruct((NPAD, F), jnp.float32),
                   jax.ShapeDtypeStruct((NPAD, F), jnp.float32)],
    )(h, lp['Wl'], lp['bl'].reshape(1, F), lp['Wr'], lp['br'].reshape(1, F))


def _post_body(h_ref, om_ref, b_ref, g_ref, bb_ref, out_ref):
    hh = om_ref[...] + b_ref[...]
    hh = jnp.maximum(hh, 0.2 * hh)
    v = h_ref[...] + hh
    m = jnp.mean(v, axis=-1, keepdims=True)
    var = jnp.mean((v - m) ** 2, axis=-1, keepdims=True)
    out_ref[...] = (v - m) / jnp.sqrt(var + 1e-5) * g_ref[...] + bb_ref[...]


def _post(h, om, bias, nrm):
    nb = N // _EB
    return pl.pallas_call(
        _post_body,
        grid=(nb,),
        in_specs=[
            pl.BlockSpec((_EB, HIDDEN), lambda i: (i, 0)),
            pl.BlockSpec((_EB, HIDDEN), lambda i: (i, 0)),
            pl.BlockSpec((1, HIDDEN), lambda i: (0, 0)),
            pl.BlockSpec((1, HIDDEN), lambda i: (0, 0)),
            pl.BlockSpec((1, HIDDEN), lambda i: (0, 0)),
        ],
        out_specs=pl.BlockSpec((_EB, HIDDEN), lambda i: (i, 0)),
        out_shape=jax.ShapeDtypeStruct((N, HIDDEN), jnp.float32),
    )(h, om, bias.reshape(1, HIDDEN), nrm['g'].reshape(1, HIDDEN),
      nrm['b'].reshape(1, HIDDEN))


_PB = 1000


def _pool_mlp_body(h_ref, nt_ref, b_ref, w_refs, acc_num, acc_cnt, out_ref):
    i = pl.program_id(0)

    @pl.when(i == 0)
    def _init():
        acc_num[...] = jnp.zeros_like(acc_num)
        acc_cnt[...] = jnp.zeros_like(acc_cnt)

    h = h_ref[...]
    nt = nt_ref[0]
    b = b_ref[0]
    mask = (nt == 0).astype(jnp.float32)
    gids = jax.lax.broadcasted_iota(jnp.int32, (NG, _PB), 0)
    onehot = (b == gids).astype(jnp.float32)
    hm = h * mask.reshape(_PB, 1)
    acc_num[...] += jnp.dot(onehot, hm, preferred_element_type=jnp.float32)
    acc_cnt[...] += jnp.dot(onehot, mask.reshape(_PB, 1),
                            preferred_element_type=jnp.float32)

    @pl.when(i == pl.num_programs(0) - 1)
    def _final():
        (W1, b1, g1, bb1, W2, b2, g2, bb2, W3, b3, g3, bb3, W4, b4) = [
            r[...] for r in w_refs]

        def ln(v, g, bb):
            m = jnp.mean(v, axis=-1, keepdims=True)
            var = jnp.mean((v - m) ** 2, axis=-1, keepdims=True)
            return (v - m) / jnp.sqrt(var + 1e-5) * g + bb

        def lrelu(v):
            return jnp.maximum(v, 0.2 * v)

        pooled = acc_num[...] / jnp.maximum(acc_cnt[...], 1.0)
        z = lrelu(ln(jnp.dot(pooled, W1, preferred_element_type=jnp.float32) + b1[0], g1[0], bb1[0]))
        z = lrelu(ln(jnp.dot(z, W2, preferred_element_type=jnp.float32) + b2[0], g2[0], bb2[0]))
        z = lrelu(ln(jnp.dot(z, W3, preferred_element_type=jnp.float32) + b3[0], g3[0], bb3[0]))
        z = jnp.dot(z, W4, preferred_element_type=jnp.float32) + b4[0]
        out_ref[...] = jnp.broadcast_to(z.reshape(1, NG), (8, NG))


def _pool_mlp(h, node_type, batch, mp):
    nb = N // _PB
    nt3 = node_type.astype(jnp.int32).reshape(nb, 1, _PB)
    b3 = batch.astype(jnp.int32).reshape(nb, 1, _PB)
    ws = [mp['W1'], mp['b1'].reshape(1, -1), mp['g1'].reshape(1, -1), mp['bb1'].reshape(1, -1),
          mp['W2'], mp['b2'].reshape(1, -1), mp['g2'].reshape(1, -1), mp['bb2'].reshape(1, -1),
          mp['W3'], mp['b3'].reshape(1, -1), mp['g3'].reshape(1, -1), mp['bb3'].reshape(1, -1),
          mp['W4'], mp['b4'].reshape(1, -1)]
    w_specs = [pl.BlockSpec(w.shape, functools.partial(lambda nd, i: (0,) * nd, w.ndim))
               for w in ws]

    grid = (nb,)
    out = pl.pallas_call(
        lambda h_ref, nt_ref, b_ref, *rest: _pool_mlp_body(
            h_ref, nt_ref, b_ref, rest[:-3], rest[-2], rest[-1], rest[-3]),
        grid=grid,
        in_specs=[
            pl.BlockSpec((_PB, HIDDEN), lambda i: (i, 0)),
            pl.BlockSpec((1, 1, _PB), lambda i: (i, 0, 0)),
            pl.BlockSpec((1, 1, _PB), lambda i: (i, 0, 0)),
            *w_specs,
        ],
        out_specs=pl.BlockSpec((8, NG), lambda i: (0, 0)),
        out_shape=jax.ShapeDtypeStruct((8, NG), jnp.float32),
        scratch_shapes=[
            pltpu.VMEM((NG, HIDDEN), jnp.float32),
            pltpu.VMEM((NG, 1), jnp.float32),
        ],
    )(h, nt3, b3, *ws)
    return out[0]


def kernel(x, node_type, edge_index, edge_attr, batch, params):
    dst_s, src_s, ea_s = jax.lax.sort(
        (edge_index[1].astype(jnp.int32), edge_index[0].astype(jnp.int32),
         edge_attr[:, 0]), num_keys=1)
    offs = jnp.searchsorted(dst_s, jnp.arange(NCHUNK + 1, dtype=jnp.int32) * CN
                            ).astype(jnp.int32)
    offs_pad = jnp.full((OFFPAD,), E, jnp.int32).at[:NCHUNK + 1].set(offs)
    zpad_i = jnp.zeros((BUF,), jnp.int32)
    src_pad = jnp.concatenate([src_s, zpad_i])
    dst_pad = jnp.concatenate([dst_s, zpad_i])
    ea_pad = jnp.concatenate([ea_s, jnp.zeros((BUF,), jnp.float32)])

    h = _embed(x, node_type, params)
    for lp, nrm in zip(params['layers'], params['norms']):
        xlw, xrw = _proj(h, lp)
        wvec = jnp.concatenate([lp['We'].reshape(F), lp['att'].reshape(F)])
        om = _edge_layer_sc(xlw, xrw, src_pad, dst_pad, ea_pad,
                            offs_pad, wvec)
        h = _post(h, om, lp['bias'], nrm)
    return _pool_mlp(h, node_type, batch, params['mlp'])
